# baseline (speedup 1.0000x reference)
import functools

import jax
import jax.numpy as jnp
from jax import lax
from jax.experimental import pallas as pl
from jax.experimental.pallas import tpu as pltpu
from jax.experimental.pallas import tpu_sc as plsc

NC, NS, LANES = 2, 16, 16
NW = NC * NS
NSLOT = 5
CH = 128


def _make_sc_lookup(R, V, D):
    assert R % (NW * CH) == 0
    RPW = R // NW
    NCH = RPW // CH
    assert NCH % NSLOT == 0 and NCH // NSLOT >= 2
    NSTAGE = 5
    assert V % NSTAGE == 0 and (V // NSTAGE) % 8 == 0
    VS = V // NSTAGE

    mesh = plsc.VectorSubcoreMesh(core_axis_name="c", subcore_axis_name="s")

    @functools.partial(
        pl.kernel,
        out_type=jax.ShapeDtypeStruct((R, D), jnp.float32),
        mesh=mesh,
        scratch_types=[
            pltpu.VMEM((NCH, CH), jnp.int32),
            pltpu.VMEM((NCH, CH), jnp.int32),
            *[pltpu.VMEM((CH, D), jnp.float32) for _ in range(NSLOT)],
            pltpu.VMEM_SHARED((V, D), jnp.float32),
            pltpu.VMEM_SHARED((V, D), jnp.float32),
            *[pltpu.SemaphoreType.DMA for _ in range(2 * NSLOT)],
        ],
    )
    def lookup(i01_hbm, w0z_hbm, w1z_hbm, out_hbm,
               i0_v, i1_v, *rest):
        bufs = rest[:NSLOT]
        t0_sh, t1_sh = rest[NSLOT], rest[NSLOT + 1]
        gsems = rest[NSLOT + 2:2 * NSLOT + 2]
        wsems = rest[2 * NSLOT + 2:]
        c = lax.axis_index("c")
        s = lax.axis_index("s")
        wid = s * NC + c

        @pl.when(s < NSTAGE)
        def _():
            pltpu.sync_copy(w0z_hbm.at[pl.ds(s * VS, VS)],
                            t0_sh.at[pl.ds(s * VS, VS)])

        @pl.when(jnp.logical_and(s >= NSTAGE, s < 2 * NSTAGE))
        def _():
            s2 = s - NSTAGE
            pltpu.sync_copy(w1z_hbm.at[pl.ds(s2 * VS, VS)],
                            t1_sh.at[pl.ds(s2 * VS, VS)])

        pltpu.sync_copy(i01_hbm.at[0, wid], i0_v)
        pltpu.sync_copy(i01_hbm.at[1, wid], i1_v)
        plsc.subcore_barrier()

        rbase = wid * RPW

        def startA(ci, b):
            src = w0z_hbm if b == NSLOT - 1 else t0_sh
            pltpu.async_copy(src.at[i0_v.at[ci]], bufs[b], gsems[b])

        def startB(ci, b):
            pltpu.async_copy(t1_sh.at[i1_v.at[ci]], bufs[b], gsems[b],
                             add=True)

        def startW(ci, b):
            pltpu.async_copy(bufs[b], out_hbm.at[pl.ds(rbase + ci * CH, CH)],
                             wsems[b])

        def waitG(b):
            pltpu.make_async_copy(t0_sh.at[i0_v.at[0]], bufs[b],
                                  gsems[b]).wait()

        def waitW(b):
            pltpu.make_async_copy(bufs[b], out_hbm.at[pl.ds(rbase, CH)],
                                  wsems[b]).wait()

        for b in range(NSLOT):
            startA(b, b)

        def group(g, _):
            for b in range(NSLOT):
                ci = g * NSLOT + b
                waitG(b)
                if b == NSLOT - 1:
                    pltpu.async_copy(w1z_hbm.at[i1_v.at[ci]], bufs[b],
                                     gsems[b], add=True)
                else:
                    startB(ci, b)
            for b in range(NSLOT):
                ci = g * NSLOT + b
                waitG(b)
                startW(ci, b)
            for b in range(NSLOT):
                ci2 = (g + 1) * NSLOT + b
                @pl.when(ci2 < NCH)
                def _():
                    waitW(b)
                    startA(ci2, b)
            return 0

        lax.fori_loop(0, NCH // NSLOT, group, 0)
        for b in range(NSLOT):
            waitW(b)

    return lookup


def kernel(idx, W0, W1):
    B, L, NB = idx.shape
    V, E = W0.shape
    assert NB == 2 and W1.shape == (V, E)
    D = NB * E
    R = B * L
    idx32 = idx.astype(jnp.int32)
    i01 = idx32.transpose(2, 1, 0).reshape(2, NW, R // (NW * CH), CH)
    w0z = jnp.pad(W0, ((0, 0), (0, E)))
    w1z = jnp.pad(W1, ((0, 0), (E, 0)))
    out = _make_sc_lookup(R, V, D)(i01, w0z, w1z)
    return out.reshape(L, B, D).transpose(1, 0, 2)


# speedup vs baseline: 1.0263x; 1.0263x over previous
import functools

import jax
import jax.numpy as jnp
from jax import lax
from jax.experimental import pallas as pl
from jax.experimental.pallas import tpu as pltpu
from jax.experimental.pallas import tpu_sc as plsc

NC, NS, LANES = 2, 16, 16
NW = NC * NS
NSLOT = 5
CH = 128


def _make_sc_lookup(R, V, D):
    assert R % (NW * CH) == 0
    RPW = R // NW
    NCH = RPW // CH
    assert NCH % NSLOT == 0 and NCH // NSLOT >= 2
    NSTAGE = 5
    assert V % NSTAGE == 0 and (V // NSTAGE) % 8 == 0
    VS = V // NSTAGE

    mesh = plsc.VectorSubcoreMesh(core_axis_name="c", subcore_axis_name="s")

    @functools.partial(
        pl.kernel,
        out_type=jax.ShapeDtypeStruct((R, D), jnp.float32),
        mesh=mesh,
        scratch_types=[
            pltpu.VMEM((NCH, CH), jnp.int32),
            pltpu.VMEM((NCH, CH), jnp.int32),
            *[pltpu.VMEM((CH, D), jnp.float32) for _ in range(NSLOT)],
            pltpu.VMEM_SHARED((V, D), jnp.float32),
            pltpu.VMEM_SHARED((V, D), jnp.float32),
            *[pltpu.SemaphoreType.DMA for _ in range(2 * NSLOT)],
        ],
    )
    def lookup(i01_hbm, w0z_hbm, w1z_hbm, out_hbm,
               i0_v, i1_v, *rest):
        bufs = rest[:NSLOT]
        t0_sh, t1_sh = rest[NSLOT], rest[NSLOT + 1]
        gsems = rest[NSLOT + 2:2 * NSLOT + 2]
        wsems = rest[2 * NSLOT + 2:]
        c = lax.axis_index("c")
        s = lax.axis_index("s")
        wid = s * NC + c

        @pl.when(s < NSTAGE)
        def _():
            pltpu.sync_copy(w0z_hbm.at[pl.ds(s * VS, VS)],
                            t0_sh.at[pl.ds(s * VS, VS)])

        @pl.when(jnp.logical_and(s >= NSTAGE, s < 2 * NSTAGE))
        def _():
            s2 = s - NSTAGE
            pltpu.sync_copy(w1z_hbm.at[pl.ds(s2 * VS, VS)],
                            t1_sh.at[pl.ds(s2 * VS, VS)])

        pltpu.sync_copy(i01_hbm.at[0, wid], i0_v)
        pltpu.sync_copy(i01_hbm.at[1, wid], i1_v)
        plsc.subcore_barrier()

        rbase = wid * RPW

        def startA(ci, b):
            src = w0z_hbm if b == NSLOT - 1 else t0_sh
            pltpu.async_copy(src.at[i0_v.at[ci]], bufs[b], gsems[b])

        def startB(ci, b):
            pltpu.async_copy(t1_sh.at[i1_v.at[ci]], bufs[b], gsems[b],
                             add=True)

        def startW(ci, b):
            pltpu.async_copy(bufs[b], out_hbm.at[pl.ds(rbase + ci * CH, CH)],
                             wsems[b])

        def waitG(b):
            pltpu.make_async_copy(t0_sh.at[i0_v.at[0]], bufs[b],
                                  gsems[b]).wait()

        def waitW(b):
            pltpu.make_async_copy(bufs[b], out_hbm.at[pl.ds(rbase, CH)],
                                  wsems[b]).wait()

        for b in range(NSLOT):
            startA(b, b)

        def group(g, _):
            for b in range(NSLOT):
                ci = g * NSLOT + b
                waitG(b)
                if b == NSLOT - 1:
                    @pl.when(g % 2 == 0)
                    def _():
                        pltpu.async_copy(w1z_hbm.at[i1_v.at[ci]], bufs[b],
                                         gsems[b], add=True)

                    @pl.when(g % 2 != 0)
                    def _():
                        startB(ci, b)
                else:
                    startB(ci, b)
            for b in range(NSLOT):
                ci = g * NSLOT + b
                waitG(b)
                startW(ci, b)
            for b in range(NSLOT):
                ci2 = (g + 1) * NSLOT + b
                @pl.when(ci2 < NCH)
                def _():
                    waitW(b)
                    startA(ci2, b)
            return 0

        lax.fori_loop(0, NCH // NSLOT, group, 0)
        for b in range(NSLOT):
            waitW(b)

    return lookup


def kernel(idx, W0, W1):
    B, L, NB = idx.shape
    V, E = W0.shape
    assert NB == 2 and W1.shape == (V, E)
    D = NB * E
    R = B * L
    idx32 = idx.astype(jnp.int32)
    i01 = idx32.transpose(2, 1, 0).reshape(2, NW, R // (NW * CH), CH)
    w0z = jnp.pad(W0, ((0, 0), (0, E)))
    w1z = jnp.pad(W1, ((0, 0), (E, 0)))
    out = _make_sc_lookup(R, V, D)(i01, w0z, w1z)
    return out.reshape(L, B, D).transpose(1, 0, 2)
